# baseline (speedup 1.0000x reference)
import functools

import jax
import jax.numpy as jnp
from jax import lax
from jax.experimental import pallas as pl
from jax.experimental.pallas import tpu as pltpu
from jax.experimental.pallas import tpu_sc as plsc

D_MODEL = 64
SCALE = 8.0

NC = 2
NS = 16
NW = NC * NS
LANES = 16

CHUNK = 128


def _make_kernel(n_idx, n_pairs):
    assert n_idx % (NW * CHUNK) == 0
    b_per_w = n_idx // NW
    steps = b_per_w // CHUNK

    mesh = plsc.VectorSubcoreMesh(core_axis_name="c", subcore_axis_name="s")

    @functools.partial(
        pl.kernel,
        out_type=jax.ShapeDtypeStruct((n_idx, D_MODEL), jnp.float32),
        mesh=mesh,
        compiler_params=pltpu.CompilerParams(use_tc_tiling_on_sc=False),
        scratch_types=[
            pltpu.VMEM((CHUNK,), jnp.int32),
            pltpu.VMEM((CHUNK,), jnp.int32),
            pltpu.VMEM((CHUNK, 2 * D_MODEL), jnp.float32),
            pltpu.VMEM((CHUNK, D_MODEL), jnp.float32),
            pltpu.SemaphoreType.DMA,
        ],
    )
    def emb_kernel(x_hbm, tab_hbm, out_hbm, idx_v, idx2_v, rows_v, out_v, gsem):
        wid = lax.axis_index("s") * NC + lax.axis_index("c")
        base = wid * b_per_w

        def halve(i, carry):
            sl = pl.ds(i * LANES, LANES)
            idx2_v[sl] = lax.shift_right_logical(idx_v[sl], 1)
            return carry

        def select_scale(g, carry):
            r0 = g * LANES
            parities = (idx_v[pl.ds(r0, LANES)] & 1) * D_MODEL
            for j in range(LANES):
                p = parities[j]
                for c in range(D_MODEL // LANES):
                    out_v[r0 + j, pl.ds(c * LANES, LANES)] = (
                        rows_v[r0 + j, pl.ds(p + c * LANES, LANES)] * SCALE
                    )
            return carry

        def step(s, carry):
            off = base + s * CHUNK
            pltpu.sync_copy(x_hbm.at[pl.ds(off, CHUNK)], idx_v)
            lax.fori_loop(0, CHUNK // LANES, halve, 0)
            pltpu.async_copy(tab_hbm.at[idx2_v], rows_v, gsem).wait()
            lax.fori_loop(0, CHUNK // LANES, select_scale, 0)
            pltpu.sync_copy(out_v, out_hbm.at[pl.ds(off, CHUNK)])
            return carry

        lax.fori_loop(0, steps, step, 0)

    return emb_kernel


def kernel(x, table):
    orig_shape = x.shape
    x_flat = x.reshape(-1).astype(jnp.int32)
    n_pairs = table.shape[0] // 2
    tab2 = table.reshape(n_pairs, 2 * D_MODEL)
    out = _make_kernel(x_flat.shape[0], n_pairs)(x_flat, tab2)
    return out.reshape(*orig_shape, D_MODEL)


# speedup vs baseline: 1.7368x; 1.7368x over previous
import functools

import jax
import jax.numpy as jnp
from jax import lax
from jax.experimental import pallas as pl
from jax.experimental.pallas import tpu as pltpu
from jax.experimental.pallas import tpu_sc as plsc

D_MODEL = 64
SCALE = 8.0

NC = 2
NS = 16
NW = NC * NS
LANES = 16

CHUNK = 128


def _make_kernel(n_idx):
    assert n_idx % (NW * CHUNK) == 0
    b_per_w = n_idx // NW
    steps = b_per_w // CHUNK

    mesh = plsc.VectorSubcoreMesh(core_axis_name="c", subcore_axis_name="s")

    @functools.partial(
        pl.kernel,
        out_type=jax.ShapeDtypeStruct((n_idx, D_MODEL), jnp.float32),
        mesh=mesh,
        compiler_params=pltpu.CompilerParams(use_tc_tiling_on_sc=True),
        scratch_types=[
            pltpu.VMEM((CHUNK,), jnp.int32),
            pltpu.VMEM((CHUNK, D_MODEL), jnp.float32),
            pltpu.SemaphoreType.DMA,
        ],
    )
    def emb_kernel(x_hbm, tab_hbm, out_hbm, idx_v, rows_v, gsem):
        wid = lax.axis_index("s") * NC + lax.axis_index("c")
        base = wid * b_per_w

        def scale_row(r, carry):
            for c in range(D_MODEL // LANES):
                sl = pl.ds(c * LANES, LANES)
                rows_v[r, sl] = rows_v[r, sl] * SCALE
            return carry

        def step(s, carry):
            off = base + s * CHUNK
            pltpu.sync_copy(x_hbm.at[pl.ds(off, CHUNK)], idx_v)
            descs = []
            for g in range(CHUNK // LANES):
                vec = idx_v[pl.ds(g * LANES, LANES)]
                for j in range(LANES):
                    r = g * LANES + j
                    descs.append(
                        pltpu.async_copy(
                            tab_hbm.at[pl.ds(vec[j], 1)],
                            rows_v.at[pl.ds(r, 1)],
                            gsem,
                        )
                    )
            for d in descs:
                d.wait()
            lax.fori_loop(0, CHUNK, scale_row, 0)
            pltpu.sync_copy(rows_v, out_hbm.at[pl.ds(off, CHUNK)])
            return carry

        lax.fori_loop(0, steps, step, 0)

    return emb_kernel


def kernel(x, table):
    orig_shape = x.shape
    x_flat = x.reshape(-1).astype(jnp.int32)
    out = _make_kernel(x_flat.shape[0])(x_flat, table)
    return out.reshape(*orig_shape, D_MODEL)


# speedup vs baseline: 2.0605x; 1.1863x over previous
import functools

import jax
import jax.numpy as jnp
from jax import lax
from jax.experimental import pallas as pl
from jax.experimental.pallas import tpu as pltpu
from jax.experimental.pallas import tpu_sc as plsc

D_MODEL = 64
SCALE = 8.0

NC = 2
NS = 16
NW = NC * NS
LANES = 16

CHUNK = 128


def _make_kernel(n_idx):
    assert n_idx % (NW * CHUNK) == 0
    b_per_w = n_idx // NW
    steps = b_per_w // CHUNK

    mesh = plsc.VectorSubcoreMesh(core_axis_name="c", subcore_axis_name="s")

    @functools.partial(
        pl.kernel,
        out_type=jax.ShapeDtypeStruct((n_idx, D_MODEL), jnp.float32),
        mesh=mesh,
        compiler_params=pltpu.CompilerParams(use_tc_tiling_on_sc=True),
        scratch_types=[
            pltpu.VMEM((CHUNK,), jnp.int32),
            pltpu.VMEM((CHUNK,), jnp.int32),
            pltpu.VMEM((CHUNK, D_MODEL), jnp.float32),
            pltpu.VMEM((CHUNK, D_MODEL), jnp.float32),
            pltpu.SemaphoreType.DMA,
            pltpu.SemaphoreType.DMA,
        ],
    )
    def emb_kernel(x_hbm, tab_hbm, out_hbm, idx0, idx1, rows0, rows1,
                   sem0, sem1):
        wid = lax.axis_index("s") * NC + lax.axis_index("c")
        base = wid * b_per_w
        idx_bufs = (idx0, idx1)
        rows_bufs = (rows0, rows1)
        sems = (sem0, sem1)

        def fetch_chunk(s, p):
            off = base + s * CHUNK
            idx_v, rows_v, sem = idx_bufs[p], rows_bufs[p], sems[p]
            pltpu.sync_copy(x_hbm.at[pl.ds(off, CHUNK)], idx_v)
            for g in range(CHUNK // LANES):
                vec = idx_v[pl.ds(g * LANES, LANES)]
                for j in range(LANES):
                    r = g * LANES + j
                    pltpu.async_copy(
                        tab_hbm.at[pl.ds(vec[j], 1)],
                        rows_v.at[pl.ds(r, 1)],
                        sem,
                    )

        def scale_row(rows_v):
            def body(r, carry):
                for c in range(D_MODEL // LANES):
                    sl = pl.ds(c * LANES, LANES)
                    rows_v[r, sl] = rows_v[r, sl] * SCALE
                return carry
            return body

        def drain(p):
            pltpu.make_async_copy(
                tab_hbm.at[pl.ds(0, CHUNK)], rows_bufs[p], sems[p]
            ).wait()

        def step(s, carry):
            p = lax.rem(s, 2)

            @pl.when(s + 1 < steps)
            def _():
                @pl.when(p == 0)
                def _():
                    fetch_chunk(s + 1, 1)

                @pl.when(p == 1)
                def _():
                    fetch_chunk(s + 1, 0)

            @pl.when(p == 0)
            def _():
                drain(0)
                lax.fori_loop(0, CHUNK, scale_row(rows0), 0)
                pltpu.sync_copy(
                    rows0, out_hbm.at[pl.ds(base + s * CHUNK, CHUNK)]
                )

            @pl.when(p == 1)
            def _():
                drain(1)
                lax.fori_loop(0, CHUNK, scale_row(rows1), 0)
                pltpu.sync_copy(
                    rows1, out_hbm.at[pl.ds(base + s * CHUNK, CHUNK)]
                )

            return carry

        fetch_chunk(0, 0)
        lax.fori_loop(0, steps, step, 0)

    return emb_kernel


def kernel(x, table):
    orig_shape = x.shape
    x_flat = x.reshape(-1).astype(jnp.int32)
    out = _make_kernel(x_flat.shape[0])(x_flat, table)
    return out.reshape(*orig_shape, D_MODEL)


# speedup vs baseline: 2.3395x; 1.1354x over previous
import functools

import jax
import jax.numpy as jnp
from jax import lax
from jax.experimental import pallas as pl
from jax.experimental.pallas import tpu as pltpu
from jax.experimental.pallas import tpu_sc as plsc

D_MODEL = 64
SCALE = 8.0

NC = 2
NS = 16
NW = NC * NS
LANES = 16

CHUNK = 128


def _make_kernel(n_idx):
    assert n_idx % (NW * CHUNK) == 0
    b_per_w = n_idx // NW
    steps = b_per_w // CHUNK

    mesh = plsc.VectorSubcoreMesh(core_axis_name="c", subcore_axis_name="s")

    @functools.partial(
        pl.kernel,
        out_type=jax.ShapeDtypeStruct((n_idx, D_MODEL), jnp.float32),
        mesh=mesh,
        compiler_params=pltpu.CompilerParams(use_tc_tiling_on_sc=True),
        scratch_types=[
            pltpu.VMEM((b_per_w,), jnp.int32),
            pltpu.VMEM((CHUNK, D_MODEL), jnp.float32),
            pltpu.VMEM((CHUNK, D_MODEL), jnp.float32),
            pltpu.SemaphoreType.DMA,
            pltpu.SemaphoreType.DMA,
            pltpu.SemaphoreType.DMA,
            pltpu.SemaphoreType.DMA,
        ],
    )
    def emb_kernel(x_hbm, tab_hbm, out_hbm, idx_all, rows0, rows1,
                   gsem0, gsem1, wsem0, wsem1):
        wid = lax.axis_index("s") * NC + lax.axis_index("c")
        base = wid * b_per_w
        rows_bufs = (rows0, rows1)
        gsems = (gsem0, gsem1)
        wsems = (wsem0, wsem1)

        def fetch_chunk(s, p):
            rows_v, sem = rows_bufs[p], gsems[p]

            def g_body(g, carry):
                vec = idx_all[pl.ds(s * CHUNK + g * LANES, LANES)]
                for j in range(LANES):
                    pltpu.async_copy(
                        tab_hbm.at[pl.ds(vec[j], 1)],
                        rows_v.at[pl.ds(g * LANES + j, 1)],
                        sem,
                    )
                return carry

            lax.fori_loop(0, CHUNK // LANES, g_body, 0)

        def process(s, p):
            rows_v = rows_bufs[p]
            pltpu.make_async_copy(
                tab_hbm.at[pl.ds(0, CHUNK)], rows_v, gsems[p]
            ).wait()

            def scale_row(r, carry):
                for c in range(D_MODEL // LANES):
                    sl = pl.ds(c * LANES, LANES)
                    rows_v[r, sl] = rows_v[r, sl] * SCALE
                return carry

            lax.fori_loop(0, CHUNK, scale_row, 0)
            pltpu.async_copy(
                rows_v, out_hbm.at[pl.ds(base + s * CHUNK, CHUNK)], wsems[p]
            )

        def wait_write(s, p):
            pltpu.make_async_copy(
                rows_bufs[p],
                out_hbm.at[pl.ds(base + s * CHUNK, CHUNK)],
                wsems[p],
            ).wait()

        def step(s, carry):
            p = lax.rem(s, 2)

            @pl.when(s + 1 < steps)
            def _():
                @pl.when(p == 0)
                def _():
                    @pl.when(s >= 1)
                    def _():
                        wait_write(s - 1, 1)
                    fetch_chunk(s + 1, 1)

                @pl.when(p == 1)
                def _():
                    wait_write(s - 1, 0)
                    fetch_chunk(s + 1, 0)

            @pl.when(p == 0)
            def _():
                process(s, 0)

            @pl.when(p == 1)
            def _():
                process(s, 1)

            return carry

        pltpu.sync_copy(x_hbm.at[pl.ds(base, b_per_w)], idx_all)
        fetch_chunk(0, 0)
        lax.fori_loop(0, steps, step, 0)
        wait_write(steps - 2, (steps - 2) % 2)
        wait_write(steps - 1, (steps - 1) % 2)

    return emb_kernel


def kernel(x, table):
    orig_shape = x.shape
    x_flat = x.reshape(-1).astype(jnp.int32)
    out = _make_kernel(x_flat.shape[0])(x_flat, table)
    return out.reshape(*orig_shape, D_MODEL)
